# baseline (speedup 1.0000x reference)
import functools

import jax
import jax.numpy as jnp
from jax import lax
from jax.experimental import pallas as pl
from jax.experimental.pallas import tpu as pltpu
from jax.experimental.pallas import tpu_sc as plsc

T = 2048
D = 768
E = 8
DFF = 3072

BT = 512
NB = T // BT
G = NB + E - 1

_SC_CORES = 2
_SC_SUBCORES = 16
_NW = _SC_CORES * _SC_SUBCORES
RPW = T // _NW


def _router_body(x_ref, wr_ref, br_ref,
                 pos_ref, loss_ref, wblk_ref, wexp_ref, wval_ref,
                 wstart_ref, wend_ref, wfirst_ref, wpf_ref, wnxt_ref,
                 wslot_ref):
    xv = x_ref[...]
    logits = jnp.dot(xv, wr_ref[...],
                     preferred_element_type=jnp.float32) + br_ref[...]
    m = jnp.max(logits, axis=1, keepdims=True)
    lane = lax.broadcasted_iota(jnp.int32, (T, E), 1)
    eidx = jnp.min(jnp.where(logits == m, lane, E), axis=1,
                   keepdims=True)
    onehot = (lane == eidx).astype(jnp.float32)

    c = onehot
    k = 1
    while k < T:
        c = c + jnp.concatenate(
            [jnp.zeros((k, E), jnp.float32), c[: T - k, :]], axis=0)
        k *= 2
    rank_excl = c - onehot
    counts = c[T - 1 : T, :]

    oc = counts
    k = 1
    while k < E:
        oc = oc + jnp.concatenate(
            [jnp.zeros((1, k), jnp.float32), oc[:, : E - k]], axis=1)
        k *= 2
    starts = oc - counts

    pos = jnp.sum(onehot * (rank_excl + starts), axis=1,
                  keepdims=True)
    pos_ref[...] = pos.astype(jnp.int32)
    usage = counts / float(T)
    loss_ref[...] = jnp.mean((usage - 1.0 / E) ** 2).reshape(1, 1)

    fb = jnp.floor(starts * (1.0 / BT))
    lb = jnp.floor((oc - 1.0) * (1.0 / BT))
    nblk = jnp.where(counts > 0, lb - fb + 1.0, 0.0)
    wse = nblk
    k = 1
    while k < E:
        wse = wse + jnp.concatenate(
            [jnp.zeros((1, k), jnp.float32), wse[:, : E - k]], axis=1)
        k *= 2
    wss = wse - nblk
    total = wse[:, E - 1 : E]

    wio = lax.broadcasted_iota(jnp.int32, (1, G), 1).astype(jnp.float32)
    weff = jnp.minimum(wio, total - 1.0)
    e_of = jnp.zeros((1, G), jnp.float32)
    for e in range(E):
        e_of = e_of + (wse[:, e : e + 1] <= weff).astype(jnp.float32)
    blk_of = jnp.zeros((1, G), jnp.float32)
    for e in range(E):
        sel = (e_of == e).astype(jnp.float32)
        blk_of = blk_of + sel * (fb[:, e : e + 1] + weff - wss[:, e : e + 1])
    wblk_ref[...] = blk_of.astype(jnp.int32)
    wexp_ref[...] = e_of.astype(jnp.int32)
    validb = wio < total
    wval_ref[...] = validb.astype(jnp.int32)
    wstart_ref[...] = starts.astype(jnp.int32)
    wend_ref[...] = oc.astype(jnp.int32)

    pe = jnp.concatenate(
        [jnp.full((1, 1), -1.0, jnp.float32), e_of[:, : G - 1]], axis=1)
    firstb = (e_of != pe) & validb
    firstf = firstb.astype(jnp.float32)
    gidx = firstf
    k = 1
    while k < G:
        gidx = gidx + jnp.concatenate(
            [jnp.zeros((1, k), jnp.float32), gidx[:, : G - k]], axis=1)
        k *= 2
    gidx = gidx - 1.0
    wslot_ref[...] = (gidx - 2.0 * jnp.floor(gidx * 0.5)).astype(jnp.int32)
    nxt = jnp.full((1, 1), -1.0, jnp.float32)
    narr = [None] * E
    for e in range(E - 1, -1, -1):
        narr[e] = nxt
        nxt = jnp.where(nblk[:, e : e + 1] > 0, float(e), nxt)
    wnexte = jnp.zeros((1, G), jnp.float32)
    for e in range(E):
        wnexte = wnexte + (e_of == e).astype(jnp.float32) * narr[e]
    wfirst_ref[...] = firstf.astype(jnp.int32)
    wpf_ref[...] = (firstb & (wnexte > -0.5)).astype(jnp.int32)
    wnxt_ref[...] = wnexte.astype(jnp.int32)


def _run_router(xf, Wr, br):
    return pl.pallas_call(
        _router_body,
        out_shape=(
            jax.ShapeDtypeStruct((T, 1), jnp.int32),
            jax.ShapeDtypeStruct((1, 1), jnp.float32),
            jax.ShapeDtypeStruct((1, G), jnp.int32),
            jax.ShapeDtypeStruct((1, G), jnp.int32),
            jax.ShapeDtypeStruct((1, G), jnp.int32),
            jax.ShapeDtypeStruct((1, E), jnp.int32),
            jax.ShapeDtypeStruct((1, E), jnp.int32),
            jax.ShapeDtypeStruct((1, G), jnp.int32),
            jax.ShapeDtypeStruct((1, G), jnp.int32),
            jax.ShapeDtypeStruct((1, G), jnp.int32),
            jax.ShapeDtypeStruct((1, G), jnp.int32),
        ),
    )(xf, Wr, br.reshape(1, E))


@functools.lru_cache(maxsize=1)
def _sc_permute_kernels():
    mesh = plsc.VectorSubcoreMesh(
        core_axis_name="c", subcore_axis_name="s",
        num_cores=_SC_CORES, num_subcores=_SC_SUBCORES)
    common = dict(
        mesh=mesh,
        out_type=jax.ShapeDtypeStruct((T, D), jnp.float32),
        scratch_types=[
            pltpu.VMEM((RPW,), jnp.int32),
            pltpu.VMEM((RPW, D), jnp.float32),
            pltpu.SemaphoreType.DMA,
        ],
    )

    @functools.partial(pl.kernel, **common)
    def sc_scatter(x_hbm, pos_hbm, out_hbm, idx_v, rows_v, sem):
        wid = lax.axis_index("s") * _SC_CORES + lax.axis_index("c")
        base = wid * RPW
        pltpu.sync_copy(pos_hbm.at[pl.ds(base, RPW)], idx_v)
        pltpu.sync_copy(x_hbm.at[pl.ds(base, RPW)], rows_v)
        pltpu.async_copy(rows_v, out_hbm.at[idx_v], sem).wait()

    @functools.partial(pl.kernel, **common)
    def sc_gather(ys_hbm, pos_hbm, out_hbm, idx_v, rows_v, sem):
        wid = lax.axis_index("s") * _SC_CORES + lax.axis_index("c")
        base = wid * RPW
        pltpu.sync_copy(pos_hbm.at[pl.ds(base, RPW)], idx_v)
        pltpu.async_copy(ys_hbm.at[idx_v], rows_v, sem).wait()
        pltpu.sync_copy(rows_v, out_hbm.at[pl.ds(base, RPW)])

    return sc_scatter, sc_gather


def _ffn_body(blk_a, e_a, valid_a, starts_a, ends_a, first_a, pf_a, nxt_a,
              slot_a, xs_ref, w1_hbm, b1_ref, w2_hbm, b2_ref, out_ref,
              w1_buf, w2_buf, sem1, sem2):
    w = pl.program_id(0)
    slot = slot_a[0, w]

    @pl.when(w == 0)
    def _():
        e0 = e_a[0, 0]
        pltpu.make_async_copy(w1_hbm.at[e0], w1_buf.at[0], sem1.at[0]).start()
        pltpu.make_async_copy(w2_hbm.at[e0], w2_buf.at[0], sem2.at[0]).start()

    @pl.when(first_a[0, w] == 1)
    def _():
        e = e_a[0, w]
        pltpu.make_async_copy(w1_hbm.at[e], w1_buf.at[slot],
                              sem1.at[slot]).wait()
        pltpu.make_async_copy(w2_hbm.at[e], w2_buf.at[slot],
                              sem2.at[slot]).wait()

    @pl.when(pf_a[0, w] == 1)
    def _():
        ne = nxt_a[0, w]
        oslot = 1 - slot
        pltpu.make_async_copy(w1_hbm.at[ne], w1_buf.at[oslot],
                              sem1.at[oslot]).start()
        pltpu.make_async_copy(w2_hbm.at[ne], w2_buf.at[oslot],
                              sem2.at[oslot]).start()

    @pl.when(valid_a[0, w] == 1)
    def _():
        e = e_a[0, w]
        xv = xs_ref[...].astype(jnp.bfloat16)
        h = jnp.dot(xv, w1_buf[slot].astype(jnp.bfloat16),
                    preferred_element_type=jnp.float32)
        h = jnp.maximum(h + b1_ref[0], 0.0).astype(jnp.bfloat16)
        y = jnp.dot(h, w2_buf[slot].astype(jnp.bfloat16),
                    preferred_element_type=jnp.float32)
        y = y + b2_ref[0]
        rows = blk_a[0, w] * BT + lax.broadcasted_iota(jnp.int32, (BT, 1), 0)
        keep = (rows >= starts_a[0, e]) & (rows < ends_a[0, e])
        out_ref[...] = jnp.where(keep, y, out_ref[...])


def _run_ffn(wblk, wexp, wval, wstart, wend, wfirst, wpf, wnxt, wslot,
             xs, W1, b1, W2, b2):
    grid_spec = pltpu.PrefetchScalarGridSpec(
        num_scalar_prefetch=9,
        grid=(G,),
        in_specs=[
            pl.BlockSpec((BT, D),
                         lambda w, blk, *_: (blk[0, w], 0)),
            pl.BlockSpec(memory_space=pltpu.MemorySpace.HBM),
            pl.BlockSpec((1, 1, DFF),
                         lambda w, blk, e, *_: (e[0, w], 0, 0)),
            pl.BlockSpec(memory_space=pltpu.MemorySpace.HBM),
            pl.BlockSpec((1, 1, D),
                         lambda w, blk, e, *_: (e[0, w], 0, 0)),
        ],
        out_specs=pl.BlockSpec((BT, D), lambda w, blk, *_: (blk[0, w], 0)),
        scratch_shapes=[
            pltpu.VMEM((2, D, DFF), jnp.float32),
            pltpu.VMEM((2, DFF, D), jnp.float32),
            pltpu.SemaphoreType.DMA((2,)),
            pltpu.SemaphoreType.DMA((2,)),
        ],
    )
    return pl.pallas_call(
        _ffn_body,
        grid_spec=grid_spec,
        out_shape=jax.ShapeDtypeStruct((T, D), jnp.float32),
        compiler_params=pltpu.CompilerParams(
            dimension_semantics=("arbitrary",)),
    )(wblk, wexp, wval, wstart, wend, wfirst, wpf, wnxt, wslot, xs,
      W1, b1.reshape(E, 1, DFF), W2, b2.reshape(E, 1, D))


def kernel(x, Wr, br, W1, b1, W2, b2):
    xf = x.reshape(T, D)
    (pos2, loss2, wblk, wexp, wval, wstart, wend,
     wfirst, wpf, wnxt, wslot) = _run_router(xf, Wr, br)
    pos = pos2.reshape(T)

    sc_scatter, sc_gather = _sc_permute_kernels()
    xs = sc_scatter(xf, pos)
    ys = _run_ffn(wblk, wexp, wval, wstart, wend, wfirst, wpf, wnxt, wslot,
                  xs, W1, b1, W2, b2)
    outf = sc_gather(ys, pos)

    return outf.reshape(x.shape), loss2.reshape(())


# speedup vs baseline: 1.0078x; 1.0078x over previous
import functools

import jax
import jax.numpy as jnp
from jax import lax
from jax.experimental import pallas as pl
from jax.experimental.pallas import tpu as pltpu
from jax.experimental.pallas import tpu_sc as plsc

T = 2048
D = 768
E = 8
DFF = 3072

BT = 128
NB = T // BT
G = NB + E - 1

_SC_CORES = 2
_SC_SUBCORES = 16
_NW = _SC_CORES * _SC_SUBCORES
RPW = T // _NW


def _router_body(x_ref, wr_ref, br_ref,
                 pos_ref, loss_ref, wblk_ref, wexp_ref, wval_ref,
                 wstart_ref, wend_ref, wfirst_ref, wpf_ref, wnxt_ref,
                 wslot_ref):
    xv = x_ref[...]
    logits = jnp.dot(xv, wr_ref[...],
                     preferred_element_type=jnp.float32) + br_ref[...]
    m = jnp.max(logits, axis=1, keepdims=True)
    lane = lax.broadcasted_iota(jnp.int32, (T, E), 1)
    eidx = jnp.min(jnp.where(logits == m, lane, E), axis=1,
                   keepdims=True)
    onehot = (lane == eidx).astype(jnp.float32)

    c = onehot
    k = 1
    while k < T:
        c = c + jnp.concatenate(
            [jnp.zeros((k, E), jnp.float32), c[: T - k, :]], axis=0)
        k *= 2
    rank_excl = c - onehot
    counts = c[T - 1 : T, :]

    oc = counts
    k = 1
    while k < E:
        oc = oc + jnp.concatenate(
            [jnp.zeros((1, k), jnp.float32), oc[:, : E - k]], axis=1)
        k *= 2
    starts = oc - counts

    pos = jnp.sum(onehot * (rank_excl + starts), axis=1,
                  keepdims=True)
    pos_ref[...] = pos.astype(jnp.int32)
    usage = counts / float(T)
    loss_ref[...] = jnp.mean((usage - 1.0 / E) ** 2).reshape(1, 1)

    fb = jnp.floor(starts * (1.0 / BT))
    lb = jnp.floor((oc - 1.0) * (1.0 / BT))
    nblk = jnp.where(counts > 0, lb - fb + 1.0, 0.0)
    wse = nblk
    k = 1
    while k < E:
        wse = wse + jnp.concatenate(
            [jnp.zeros((1, k), jnp.float32), wse[:, : E - k]], axis=1)
        k *= 2
    wss = wse - nblk
    total = wse[:, E - 1 : E]

    wio = lax.broadcasted_iota(jnp.int32, (1, G), 1).astype(jnp.float32)
    weff = jnp.minimum(wio, total - 1.0)
    e_of = jnp.zeros((1, G), jnp.float32)
    for e in range(E):
        e_of = e_of + (wse[:, e : e + 1] <= weff).astype(jnp.float32)
    blk_of = jnp.zeros((1, G), jnp.float32)
    for e in range(E):
        sel = (e_of == e).astype(jnp.float32)
        blk_of = blk_of + sel * (fb[:, e : e + 1] + weff - wss[:, e : e + 1])
    wblk_ref[...] = blk_of.astype(jnp.int32)
    wexp_ref[...] = e_of.astype(jnp.int32)
    validb = wio < total
    wval_ref[...] = validb.astype(jnp.int32)
    wstart_ref[...] = starts.astype(jnp.int32)
    wend_ref[...] = oc.astype(jnp.int32)

    pe = jnp.concatenate(
        [jnp.full((1, 1), -1.0, jnp.float32), e_of[:, : G - 1]], axis=1)
    firstb = (e_of != pe) & validb
    firstf = firstb.astype(jnp.float32)
    gidx = firstf
    k = 1
    while k < G:
        gidx = gidx + jnp.concatenate(
            [jnp.zeros((1, k), jnp.float32), gidx[:, : G - k]], axis=1)
        k *= 2
    gidx = gidx - 1.0
    wslot_ref[...] = (gidx - 2.0 * jnp.floor(gidx * 0.5)).astype(jnp.int32)
    nxt = jnp.full((1, 1), -1.0, jnp.float32)
    narr = [None] * E
    for e in range(E - 1, -1, -1):
        narr[e] = nxt
        nxt = jnp.where(nblk[:, e : e + 1] > 0, float(e), nxt)
    wnexte = jnp.zeros((1, G), jnp.float32)
    for e in range(E):
        wnexte = wnexte + (e_of == e).astype(jnp.float32) * narr[e]
    wfirst_ref[...] = firstf.astype(jnp.int32)
    wpf_ref[...] = (firstb & (wnexte > -0.5)).astype(jnp.int32)
    wnxt_ref[...] = wnexte.astype(jnp.int32)


def _run_router(xf, Wr, br):
    return pl.pallas_call(
        _router_body,
        out_shape=(
            jax.ShapeDtypeStruct((T, 1), jnp.int32),
            jax.ShapeDtypeStruct((1, 1), jnp.float32),
            jax.ShapeDtypeStruct((1, G), jnp.int32),
            jax.ShapeDtypeStruct((1, G), jnp.int32),
            jax.ShapeDtypeStruct((1, G), jnp.int32),
            jax.ShapeDtypeStruct((1, E), jnp.int32),
            jax.ShapeDtypeStruct((1, E), jnp.int32),
            jax.ShapeDtypeStruct((1, G), jnp.int32),
            jax.ShapeDtypeStruct((1, G), jnp.int32),
            jax.ShapeDtypeStruct((1, G), jnp.int32),
            jax.ShapeDtypeStruct((1, G), jnp.int32),
        ),
    )(xf, Wr, br.reshape(1, E))


@functools.lru_cache(maxsize=1)
def _sc_permute_kernels():
    mesh = plsc.VectorSubcoreMesh(
        core_axis_name="c", subcore_axis_name="s",
        num_cores=_SC_CORES, num_subcores=_SC_SUBCORES)
    common = dict(
        mesh=mesh,
        out_type=jax.ShapeDtypeStruct((T, D), jnp.float32),
        scratch_types=[
            pltpu.VMEM((RPW,), jnp.int32),
            pltpu.VMEM((RPW, D), jnp.float32),
            pltpu.SemaphoreType.DMA,
        ],
    )

    @functools.partial(pl.kernel, **common)
    def sc_scatter(x_hbm, pos_hbm, out_hbm, idx_v, rows_v, sem):
        wid = lax.axis_index("s") * _SC_CORES + lax.axis_index("c")
        base = wid * RPW
        pltpu.sync_copy(pos_hbm.at[pl.ds(base, RPW)], idx_v)
        pltpu.sync_copy(x_hbm.at[pl.ds(base, RPW)], rows_v)
        pltpu.async_copy(rows_v, out_hbm.at[idx_v], sem).wait()

    @functools.partial(pl.kernel, **common)
    def sc_gather(ys_hbm, pos_hbm, out_hbm, idx_v, rows_v, sem):
        wid = lax.axis_index("s") * _SC_CORES + lax.axis_index("c")
        base = wid * RPW
        pltpu.sync_copy(pos_hbm.at[pl.ds(base, RPW)], idx_v)
        pltpu.async_copy(ys_hbm.at[idx_v], rows_v, sem).wait()
        pltpu.sync_copy(rows_v, out_hbm.at[pl.ds(base, RPW)])

    return sc_scatter, sc_gather


def _ffn_body(blk_a, e_a, valid_a, starts_a, ends_a, first_a, pf_a, nxt_a,
              slot_a, xs_ref, w1_hbm, b1_ref, w2_hbm, b2_ref, out_ref,
              w1_buf, w2_buf, sem1, sem2):
    w = pl.program_id(0)
    slot = slot_a[0, w]

    @pl.when(w == 0)
    def _():
        e0 = e_a[0, 0]
        pltpu.make_async_copy(w1_hbm.at[e0], w1_buf.at[0], sem1.at[0]).start()
        pltpu.make_async_copy(w2_hbm.at[e0], w2_buf.at[0], sem2.at[0]).start()

    @pl.when(first_a[0, w] == 1)
    def _():
        e = e_a[0, w]
        pltpu.make_async_copy(w1_hbm.at[e], w1_buf.at[slot],
                              sem1.at[slot]).wait()
        pltpu.make_async_copy(w2_hbm.at[e], w2_buf.at[slot],
                              sem2.at[slot]).wait()

    @pl.when(pf_a[0, w] == 1)
    def _():
        ne = nxt_a[0, w]
        oslot = 1 - slot
        pltpu.make_async_copy(w1_hbm.at[ne], w1_buf.at[oslot],
                              sem1.at[oslot]).start()
        pltpu.make_async_copy(w2_hbm.at[ne], w2_buf.at[oslot],
                              sem2.at[oslot]).start()

    @pl.when(valid_a[0, w] == 1)
    def _():
        e = e_a[0, w]
        xv = xs_ref[...].astype(jnp.bfloat16)
        h = jnp.dot(xv, w1_buf[slot].astype(jnp.bfloat16),
                    preferred_element_type=jnp.float32)
        h = jnp.maximum(h + b1_ref[0], 0.0).astype(jnp.bfloat16)
        y = jnp.dot(h, w2_buf[slot].astype(jnp.bfloat16),
                    preferred_element_type=jnp.float32)
        y = y + b2_ref[0]
        rows = blk_a[0, w] * BT + lax.broadcasted_iota(jnp.int32, (BT, 1), 0)
        keep = (rows >= starts_a[0, e]) & (rows < ends_a[0, e])
        out_ref[...] = jnp.where(keep, y, out_ref[...])


def _run_ffn(wblk, wexp, wval, wstart, wend, wfirst, wpf, wnxt, wslot,
             xs, W1, b1, W2, b2):
    grid_spec = pltpu.PrefetchScalarGridSpec(
        num_scalar_prefetch=9,
        grid=(G,),
        in_specs=[
            pl.BlockSpec((BT, D),
                         lambda w, blk, *_: (blk[0, w], 0)),
            pl.BlockSpec(memory_space=pltpu.MemorySpace.HBM),
            pl.BlockSpec((1, 1, DFF),
                         lambda w, blk, e, *_: (e[0, w], 0, 0)),
            pl.BlockSpec(memory_space=pltpu.MemorySpace.HBM),
            pl.BlockSpec((1, 1, D),
                         lambda w, blk, e, *_: (e[0, w], 0, 0)),
        ],
        out_specs=pl.BlockSpec((BT, D), lambda w, blk, *_: (blk[0, w], 0)),
        scratch_shapes=[
            pltpu.VMEM((2, D, DFF), jnp.float32),
            pltpu.VMEM((2, DFF, D), jnp.float32),
            pltpu.SemaphoreType.DMA((2,)),
            pltpu.SemaphoreType.DMA((2,)),
        ],
    )
    return pl.pallas_call(
        _ffn_body,
        grid_spec=grid_spec,
        out_shape=jax.ShapeDtypeStruct((T, D), jnp.float32),
        compiler_params=pltpu.CompilerParams(
            dimension_semantics=("arbitrary",)),
    )(wblk, wexp, wval, wstart, wend, wfirst, wpf, wnxt, wslot, xs,
      W1, b1.reshape(E, 1, DFF), W2, b2.reshape(E, 1, D))


def kernel(x, Wr, br, W1, b1, W2, b2):
    xf = x.reshape(T, D)
    (pos2, loss2, wblk, wexp, wval, wstart, wend,
     wfirst, wpf, wnxt, wslot) = _run_router(xf, Wr, br)
    pos = pos2.reshape(T)

    sc_scatter, sc_gather = _sc_permute_kernels()
    xs = sc_scatter(xf, pos)
    ys = _run_ffn(wblk, wexp, wval, wstart, wend, wfirst, wpf, wnxt, wslot,
                  xs, W1, b1, W2, b2)
    outf = sc_gather(ys, pos)

    return outf.reshape(x.shape), loss2.reshape(())


# speedup vs baseline: 1.1353x; 1.1264x over previous
import functools

import jax
import jax.numpy as jnp
from jax import lax
from jax.experimental import pallas as pl
from jax.experimental.pallas import tpu as pltpu
from jax.experimental.pallas import tpu_sc as plsc

T = 2048
D = 768
E = 8
DFF = 3072

BT = 256
NB = T // BT
G = NB + E - 1

_SC_CORES = 2
_SC_SUBCORES = 16
_NW = _SC_CORES * _SC_SUBCORES
RPW = T // _NW


def _router_body(x_ref, wr_ref, br_ref,
                 pos_ref, loss_ref, wblk_ref, wexp_ref, wval_ref,
                 wstart_ref, wend_ref, wfirst_ref, wpf_ref, wnxt_ref,
                 wslot_ref, wslot3_ref, wpf2_ref, wnxt2_ref):
    xv = x_ref[...]
    logits = jnp.dot(xv, wr_ref[...],
                     preferred_element_type=jnp.float32) + br_ref[...]
    m = jnp.max(logits, axis=1, keepdims=True)
    lane = lax.broadcasted_iota(jnp.int32, (T, E), 1)
    eidx = jnp.min(jnp.where(logits == m, lane, E), axis=1,
                   keepdims=True)
    onehot = (lane == eidx).astype(jnp.float32)

    c = onehot
    k = 1
    while k < T:
        c = c + jnp.concatenate(
            [jnp.zeros((k, E), jnp.float32), c[: T - k, :]], axis=0)
        k *= 2
    rank_excl = c - onehot
    counts = c[T - 1 : T, :]

    oc = counts
    k = 1
    while k < E:
        oc = oc + jnp.concatenate(
            [jnp.zeros((1, k), jnp.float32), oc[:, : E - k]], axis=1)
        k *= 2
    starts = oc - counts

    pos = jnp.sum(onehot * (rank_excl + starts), axis=1,
                  keepdims=True)
    pos_ref[...] = pos.astype(jnp.int32)
    usage = counts / float(T)
    loss_ref[...] = jnp.mean((usage - 1.0 / E) ** 2).reshape(1, 1)

    fb = jnp.floor(starts * (1.0 / BT))
    lb = jnp.floor((oc - 1.0) * (1.0 / BT))
    nblk = jnp.where(counts > 0, lb - fb + 1.0, 0.0)
    wse = nblk
    k = 1
    while k < E:
        wse = wse + jnp.concatenate(
            [jnp.zeros((1, k), jnp.float32), wse[:, : E - k]], axis=1)
        k *= 2
    wss = wse - nblk
    total = wse[:, E - 1 : E]

    wio = lax.broadcasted_iota(jnp.int32, (1, G), 1).astype(jnp.float32)
    weff = jnp.minimum(wio, total - 1.0)
    e_of = jnp.zeros((1, G), jnp.float32)
    for e in range(E):
        e_of = e_of + (wse[:, e : e + 1] <= weff).astype(jnp.float32)
    blk_of = jnp.zeros((1, G), jnp.float32)
    for e in range(E):
        sel = (e_of == e).astype(jnp.float32)
        blk_of = blk_of + sel * (fb[:, e : e + 1] + weff - wss[:, e : e + 1])
    wblk_ref[...] = blk_of.astype(jnp.int32)
    wexp_ref[...] = e_of.astype(jnp.int32)
    validb = wio < total
    wval_ref[...] = validb.astype(jnp.int32)
    wstart_ref[...] = starts.astype(jnp.int32)
    wend_ref[...] = oc.astype(jnp.int32)

    pe = jnp.concatenate(
        [jnp.full((1, 1), -1.0, jnp.float32), e_of[:, : G - 1]], axis=1)
    firstb = (e_of != pe) & validb
    firstf = firstb.astype(jnp.float32)
    gidx = firstf
    k = 1
    while k < G:
        gidx = gidx + jnp.concatenate(
            [jnp.zeros((1, k), jnp.float32), gidx[:, : G - k]], axis=1)
        k *= 2
    gidx = gidx - 1.0
    wslot_ref[...] = (gidx - 2.0 * jnp.floor(gidx * 0.5)).astype(jnp.int32)
    wslot3_ref[...] = (gidx - 3.0 * jnp.floor(gidx * (1.0 / 3.0))).astype(
        jnp.int32)
    nxt1 = jnp.full((1, 1), -1.0, jnp.float32)
    nxt2 = jnp.full((1, 1), -1.0, jnp.float32)
    narr = [None] * E
    narr2 = [None] * E
    for e in range(E - 1, -1, -1):
        narr[e] = nxt1
        narr2[e] = nxt2
        act = nblk[:, e : e + 1] > 0
        nxt2 = jnp.where(act, nxt1, nxt2)
        nxt1 = jnp.where(act, float(e), nxt1)
    wnexte = jnp.zeros((1, G), jnp.float32)
    wnexte2 = jnp.zeros((1, G), jnp.float32)
    for e in range(E):
        sel = (e_of == e).astype(jnp.float32)
        wnexte = wnexte + sel * narr[e]
        wnexte2 = wnexte2 + sel * narr2[e]
    wfirst_ref[...] = firstf.astype(jnp.int32)
    wpf_ref[...] = (firstb & (wnexte > -0.5)).astype(jnp.int32)
    wnxt_ref[...] = wnexte.astype(jnp.int32)
    wpf2_ref[...] = (firstb & (wnexte2 > -0.5)).astype(jnp.int32)
    wnxt2_ref[...] = wnexte2.astype(jnp.int32)


def _run_router(xf, Wr, br):
    return pl.pallas_call(
        _router_body,
        out_shape=(
            jax.ShapeDtypeStruct((T, 1), jnp.int32),
            jax.ShapeDtypeStruct((1, 1), jnp.float32),
            jax.ShapeDtypeStruct((1, G), jnp.int32),
            jax.ShapeDtypeStruct((1, G), jnp.int32),
            jax.ShapeDtypeStruct((1, G), jnp.int32),
            jax.ShapeDtypeStruct((1, E), jnp.int32),
            jax.ShapeDtypeStruct((1, E), jnp.int32),
            jax.ShapeDtypeStruct((1, G), jnp.int32),
            jax.ShapeDtypeStruct((1, G), jnp.int32),
            jax.ShapeDtypeStruct((1, G), jnp.int32),
            jax.ShapeDtypeStruct((1, G), jnp.int32),
            jax.ShapeDtypeStruct((1, G), jnp.int32),
            jax.ShapeDtypeStruct((1, G), jnp.int32),
            jax.ShapeDtypeStruct((1, G), jnp.int32),
        ),
    )(xf, Wr, br.reshape(1, E))


@functools.lru_cache(maxsize=1)
def _sc_permute_kernels():
    mesh = plsc.VectorSubcoreMesh(
        core_axis_name="c", subcore_axis_name="s",
        num_cores=_SC_CORES, num_subcores=_SC_SUBCORES)
    common = dict(
        mesh=mesh,
        out_type=jax.ShapeDtypeStruct((T, D), jnp.float32),
        scratch_types=[
            pltpu.VMEM((RPW,), jnp.int32),
            pltpu.VMEM((RPW, D), jnp.float32),
            pltpu.SemaphoreType.DMA,
        ],
    )

    @functools.partial(pl.kernel, **common)
    def sc_scatter(x_hbm, pos_hbm, out_hbm, idx_v, rows_v, sem):
        wid = lax.axis_index("s") * _SC_CORES + lax.axis_index("c")
        base = wid * RPW
        pltpu.sync_copy(pos_hbm.at[pl.ds(base, RPW)], idx_v)
        pltpu.sync_copy(x_hbm.at[pl.ds(base, RPW)], rows_v)
        pltpu.async_copy(rows_v, out_hbm.at[idx_v], sem).wait()

    @functools.partial(pl.kernel, **common)
    def sc_gather(ys_hbm, pos_hbm, out_hbm, idx_v, rows_v, sem):
        wid = lax.axis_index("s") * _SC_CORES + lax.axis_index("c")
        base = wid * RPW
        pltpu.sync_copy(pos_hbm.at[pl.ds(base, RPW)], idx_v)
        pltpu.async_copy(ys_hbm.at[idx_v], rows_v, sem).wait()
        pltpu.sync_copy(rows_v, out_hbm.at[pl.ds(base, RPW)])

    return sc_scatter, sc_gather


def _ffn_body(blk_a, e_a, valid_a, starts_a, ends_a, first_a, pf_a, nxt_a,
              slot_a, slot3_a, pf2_a, nxt2_a,
              xs_ref, w1_hbm, b1_ref, w2_hbm, b2_ref, out_ref,
              w1_buf, w2_buf, sem1, sem2):
    w = pl.program_id(0)
    slot = slot_a[0, w]
    slot3 = slot3_a[0, w]

    @pl.when(w == 0)
    def _():
        e0 = e_a[0, 0]
        pltpu.make_async_copy(w1_hbm.at[e0], w1_buf.at[0], sem1.at[0]).start()
        pltpu.make_async_copy(w2_hbm.at[e0], w2_buf.at[0], sem2.at[0]).start()

        @pl.when(pf_a[0, 0] == 1)
        def _():
            n1 = nxt_a[0, 0]
            pltpu.make_async_copy(w1_hbm.at[n1], w1_buf.at[1],
                                  sem1.at[1]).start()

    @pl.when(first_a[0, w] == 1)
    def _():
        e = e_a[0, w]
        pltpu.make_async_copy(w1_hbm.at[e], w1_buf.at[slot3],
                              sem1.at[slot3]).wait()
        pltpu.make_async_copy(w2_hbm.at[e], w2_buf.at[slot],
                              sem2.at[slot]).wait()

    @pl.when(pf_a[0, w] == 1)
    def _():
        ne = nxt_a[0, w]
        oslot = 1 - slot
        pltpu.make_async_copy(w2_hbm.at[ne], w2_buf.at[oslot],
                              sem2.at[oslot]).start()

    @pl.when(pf2_a[0, w] == 1)
    def _():
        ne2 = nxt2_a[0, w]
        os3 = slot3 - 1
        os3 = jnp.where(os3 < 0, os3 + 3, os3)
        pltpu.make_async_copy(w1_hbm.at[ne2], w1_buf.at[os3],
                              sem1.at[os3]).start()

    @pl.when(valid_a[0, w] == 1)
    def _():
        e = e_a[0, w]
        xv = xs_ref[...].astype(jnp.bfloat16)
        h = jnp.dot(xv, w1_buf[slot3].astype(jnp.bfloat16),
                    preferred_element_type=jnp.float32)
        h = jnp.maximum(h + b1_ref[0], 0.0).astype(jnp.bfloat16)
        y = jnp.dot(h, w2_buf[slot].astype(jnp.bfloat16),
                    preferred_element_type=jnp.float32)
        y = y + b2_ref[0]
        rows = blk_a[0, w] * BT + lax.broadcasted_iota(jnp.int32, (BT, 1), 0)
        keep = (rows >= starts_a[0, e]) & (rows < ends_a[0, e])
        out_ref[...] = jnp.where(keep, y, out_ref[...])


def _run_ffn(wblk, wexp, wval, wstart, wend, wfirst, wpf, wnxt, wslot,
             wslot3, wpf2, wnxt2, xs, W1, b1, W2, b2):
    grid_spec = pltpu.PrefetchScalarGridSpec(
        num_scalar_prefetch=12,
        grid=(G,),
        in_specs=[
            pl.BlockSpec((BT, D),
                         lambda w, blk, *_: (blk[0, w], 0)),
            pl.BlockSpec(memory_space=pltpu.MemorySpace.HBM),
            pl.BlockSpec((1, 1, DFF),
                         lambda w, blk, e, *_: (e[0, w], 0, 0)),
            pl.BlockSpec(memory_space=pltpu.MemorySpace.HBM),
            pl.BlockSpec((1, 1, D),
                         lambda w, blk, e, *_: (e[0, w], 0, 0)),
        ],
        out_specs=pl.BlockSpec((BT, D), lambda w, blk, *_: (blk[0, w], 0)),
        scratch_shapes=[
            pltpu.VMEM((3, D, DFF), jnp.float32),
            pltpu.VMEM((2, DFF, D), jnp.float32),
            pltpu.SemaphoreType.DMA((3,)),
            pltpu.SemaphoreType.DMA((2,)),
        ],
    )
    return pl.pallas_call(
        _ffn_body,
        grid_spec=grid_spec,
        out_shape=jax.ShapeDtypeStruct((T, D), jnp.float32),
        compiler_params=pltpu.CompilerParams(
            dimension_semantics=("arbitrary",)),
    )(wblk, wexp, wval, wstart, wend, wfirst, wpf, wnxt, wslot,
      wslot3, wpf2, wnxt2, xs,
      W1, b1.reshape(E, 1, DFF), W2, b2.reshape(E, 1, D))


def kernel(x, Wr, br, W1, b1, W2, b2):
    xf = x.reshape(T, D)
    (pos2, loss2, wblk, wexp, wval, wstart, wend,
     wfirst, wpf, wnxt, wslot, wslot3, wpf2, wnxt2) = _run_router(xf, Wr, br)
    pos = pos2.reshape(T)

    sc_scatter, sc_gather = _sc_permute_kernels()
    xs = sc_scatter(xf, pos)
    ys = _run_ffn(wblk, wexp, wval, wstart, wend, wfirst, wpf, wnxt, wslot,
                  wslot3, wpf2, wnxt2, xs, W1, b1, W2, b2)
    outf = sc_gather(ys, pos)

    return outf.reshape(x.shape), loss2.reshape(())


# speedup vs baseline: 1.1489x; 1.0120x over previous
import functools

import jax
import jax.numpy as jnp
from jax import lax
from jax.experimental import pallas as pl
from jax.experimental.pallas import tpu as pltpu
from jax.experimental.pallas import tpu_sc as plsc

T = 2048
D = 768
E = 8
DFF = 3072

BT = 256
NB = T // BT
G = NB + E - 1

_SC_CORES = 2
_SC_SUBCORES = 16
_NW = _SC_CORES * _SC_SUBCORES
RPW = T // _NW


def _router_body(x_ref, wr_ref, br_ref,
                 pos_ref, loss_ref, wblk_ref, wexp_ref, wval_ref,
                 wstart_ref, wend_ref, wfirst_ref, wpf_ref, wnxt_ref,
                 wslot_ref):
    xv = x_ref[...]
    logits = jnp.dot(xv, wr_ref[...],
                     preferred_element_type=jnp.float32) + br_ref[...]
    m = jnp.max(logits, axis=1, keepdims=True)
    lane = lax.broadcasted_iota(jnp.int32, (T, E), 1)
    eidx = jnp.min(jnp.where(logits == m, lane, E), axis=1,
                   keepdims=True)
    onehot = (lane == eidx).astype(jnp.float32)

    c = onehot
    k = 1
    while k < T:
        c = c + jnp.concatenate(
            [jnp.zeros((k, E), jnp.float32), c[: T - k, :]], axis=0)
        k *= 2
    rank_excl = c - onehot
    counts = c[T - 1 : T, :]

    oc = counts
    k = 1
    while k < E:
        oc = oc + jnp.concatenate(
            [jnp.zeros((1, k), jnp.float32), oc[:, : E - k]], axis=1)
        k *= 2
    starts = oc - counts

    pos = jnp.sum(onehot * (rank_excl + starts), axis=1,
                  keepdims=True)
    pos_ref[...] = pos.astype(jnp.int32)
    usage = counts / float(T)
    loss_ref[...] = jnp.mean((usage - 1.0 / E) ** 2).reshape(1, 1)

    fb = jnp.floor(starts * (1.0 / BT))
    lb = jnp.floor((oc - 1.0) * (1.0 / BT))
    nblk = jnp.where(counts > 0, lb - fb + 1.0, 0.0)
    wse = nblk
    k = 1
    while k < E:
        wse = wse + jnp.concatenate(
            [jnp.zeros((1, k), jnp.float32), wse[:, : E - k]], axis=1)
        k *= 2
    wss = wse - nblk
    total = wse[:, E - 1 : E]

    wio = lax.broadcasted_iota(jnp.int32, (1, G), 1).astype(jnp.float32)
    weff = jnp.minimum(wio, total - 1.0)
    e_of = jnp.zeros((1, G), jnp.float32)
    for e in range(E):
        e_of = e_of + (wse[:, e : e + 1] <= weff).astype(jnp.float32)
    blk_of = jnp.zeros((1, G), jnp.float32)
    for e in range(E):
        sel = (e_of == e).astype(jnp.float32)
        blk_of = blk_of + sel * (fb[:, e : e + 1] + weff - wss[:, e : e + 1])
    wblk_ref[...] = blk_of.astype(jnp.int32)
    wexp_ref[...] = e_of.astype(jnp.int32)
    validb = wio < total
    wval_ref[...] = validb.astype(jnp.int32)
    wstart_ref[...] = starts.astype(jnp.int32)
    wend_ref[...] = oc.astype(jnp.int32)

    pe = jnp.concatenate(
        [jnp.full((1, 1), -1.0, jnp.float32), e_of[:, : G - 1]], axis=1)
    firstb = (e_of != pe) & validb
    firstf = firstb.astype(jnp.float32)
    gidx = firstf
    k = 1
    while k < G:
        gidx = gidx + jnp.concatenate(
            [jnp.zeros((1, k), jnp.float32), gidx[:, : G - k]], axis=1)
        k *= 2
    gidx = gidx - 1.0
    wslot_ref[...] = (gidx - 2.0 * jnp.floor(gidx * 0.5)).astype(jnp.int32)
    nxt = jnp.full((1, 1), -1.0, jnp.float32)
    narr = [None] * E
    for e in range(E - 1, -1, -1):
        narr[e] = nxt
        nxt = jnp.where(nblk[:, e : e + 1] > 0, float(e), nxt)
    wnexte = jnp.zeros((1, G), jnp.float32)
    for e in range(E):
        wnexte = wnexte + (e_of == e).astype(jnp.float32) * narr[e]
    wfirst_ref[...] = firstf.astype(jnp.int32)
    wpf_ref[...] = (firstb & (wnexte > -0.5)).astype(jnp.int32)
    wnxt_ref[...] = wnexte.astype(jnp.int32)


def _run_router(xf, Wr, br):
    return pl.pallas_call(
        _router_body,
        out_shape=(
            jax.ShapeDtypeStruct((T, 1), jnp.int32),
            jax.ShapeDtypeStruct((1, 1), jnp.float32),
            jax.ShapeDtypeStruct((1, G), jnp.int32),
            jax.ShapeDtypeStruct((1, G), jnp.int32),
            jax.ShapeDtypeStruct((1, G), jnp.int32),
            jax.ShapeDtypeStruct((1, E), jnp.int32),
            jax.ShapeDtypeStruct((1, E), jnp.int32),
            jax.ShapeDtypeStruct((1, G), jnp.int32),
            jax.ShapeDtypeStruct((1, G), jnp.int32),
            jax.ShapeDtypeStruct((1, G), jnp.int32),
            jax.ShapeDtypeStruct((1, G), jnp.int32),
        ),
    )(xf, Wr, br.reshape(1, E))


@functools.lru_cache(maxsize=1)
def _sc_permute_kernels():
    mesh = plsc.VectorSubcoreMesh(
        core_axis_name="c", subcore_axis_name="s",
        num_cores=_SC_CORES, num_subcores=_SC_SUBCORES)
    common = dict(
        mesh=mesh,
        out_type=jax.ShapeDtypeStruct((T, D), jnp.float32),
        scratch_types=[
            pltpu.VMEM((RPW,), jnp.int32),
            pltpu.VMEM((RPW, D), jnp.float32),
            pltpu.SemaphoreType.DMA,
        ],
    )

    @functools.partial(pl.kernel, **common)
    def sc_scatter(x_hbm, pos_hbm, out_hbm, idx_v, rows_v, sem):
        wid = lax.axis_index("s") * _SC_CORES + lax.axis_index("c")
        base = wid * RPW
        pltpu.sync_copy(pos_hbm.at[pl.ds(base, RPW)], idx_v)
        pltpu.sync_copy(x_hbm.at[pl.ds(base, RPW)], rows_v)
        pltpu.async_copy(rows_v, out_hbm.at[idx_v], sem).wait()

    @functools.partial(pl.kernel, **common)
    def sc_gather(ys_hbm, pos_hbm, out_hbm, idx_v, rows_v, sem):
        wid = lax.axis_index("s") * _SC_CORES + lax.axis_index("c")
        base = wid * RPW
        pltpu.sync_copy(pos_hbm.at[pl.ds(base, RPW)], idx_v)
        pltpu.async_copy(ys_hbm.at[idx_v], rows_v, sem).wait()
        pltpu.sync_copy(rows_v, out_hbm.at[pl.ds(base, RPW)])

    return sc_scatter, sc_gather


def _ffn_body(blk_a, e_a, valid_a, starts_a, ends_a, first_a, pf_a, nxt_a,
              slot_a, xs_ref, w1_hbm, b1_ref, w2_hbm, b2_ref, out_ref,
              w1_buf, w2_buf, sem1, sem2):
    w = pl.program_id(0)
    slot = slot_a[0, w]

    @pl.when(w == 0)
    def _():
        e0 = e_a[0, 0]
        pltpu.make_async_copy(w1_hbm.at[e0], w1_buf.at[0], sem1.at[0]).start()
        pltpu.make_async_copy(w2_hbm.at[e0], w2_buf.at[0], sem2.at[0]).start()

    @pl.when(first_a[0, w] == 1)
    def _():
        e = e_a[0, w]
        pltpu.make_async_copy(w1_hbm.at[e], w1_buf.at[slot],
                              sem1.at[slot]).wait()
        pltpu.make_async_copy(w2_hbm.at[e], w2_buf.at[slot],
                              sem2.at[slot]).wait()

    @pl.when(pf_a[0, w] == 1)
    def _():
        ne = nxt_a[0, w]
        oslot = 1 - slot
        pltpu.make_async_copy(w1_hbm.at[ne], w1_buf.at[oslot],
                              sem1.at[oslot]).start()
        pltpu.make_async_copy(w2_hbm.at[ne], w2_buf.at[oslot],
                              sem2.at[oslot]).start()

    @pl.when(valid_a[0, w] == 1)
    def _():
        e = e_a[0, w]
        xv = xs_ref[...].astype(jnp.bfloat16)
        h = jnp.dot(xv, w1_buf[slot].astype(jnp.bfloat16),
                    preferred_element_type=jnp.float32)
        h = jnp.maximum(h + b1_ref[0], 0.0).astype(jnp.bfloat16)
        y = jnp.dot(h, w2_buf[slot].astype(jnp.bfloat16),
                    preferred_element_type=jnp.float32)
        y = y + b2_ref[0]
        rows = blk_a[0, w] * BT + lax.broadcasted_iota(jnp.int32, (BT, 1), 0)
        keep = (rows >= starts_a[0, e]) & (rows < ends_a[0, e])
        out_ref[...] = jnp.where(keep, y, out_ref[...])


def _run_ffn(wblk, wexp, wval, wstart, wend, wfirst, wpf, wnxt, wslot,
             xs, W1, b1, W2, b2):
    grid_spec = pltpu.PrefetchScalarGridSpec(
        num_scalar_prefetch=9,
        grid=(G,),
        in_specs=[
            pl.BlockSpec((BT, D),
                         lambda w, blk, *_: (blk[0, w], 0)),
            pl.BlockSpec(memory_space=pltpu.MemorySpace.HBM),
            pl.BlockSpec((1, 1, DFF),
                         lambda w, blk, e, *_: (e[0, w], 0, 0)),
            pl.BlockSpec(memory_space=pltpu.MemorySpace.HBM),
            pl.BlockSpec((1, 1, D),
                         lambda w, blk, e, *_: (e[0, w], 0, 0)),
        ],
        out_specs=pl.BlockSpec((BT, D), lambda w, blk, *_: (blk[0, w], 0)),
        scratch_shapes=[
            pltpu.VMEM((2, D, DFF), jnp.float32),
            pltpu.VMEM((2, DFF, D), jnp.float32),
            pltpu.SemaphoreType.DMA((2,)),
            pltpu.SemaphoreType.DMA((2,)),
        ],
    )
    return pl.pallas_call(
        _ffn_body,
        grid_spec=grid_spec,
        out_shape=jax.ShapeDtypeStruct((T, D), jnp.float32),
        compiler_params=pltpu.CompilerParams(
            dimension_semantics=("arbitrary",)),
    )(wblk, wexp, wval, wstart, wend, wfirst, wpf, wnxt, wslot, xs,
      W1, b1.reshape(E, 1, DFF), W2, b2.reshape(E, 1, D))


def kernel(x, Wr, br, W1, b1, W2, b2):
    xf = x.reshape(T, D)
    (pos2, loss2, wblk, wexp, wval, wstart, wend,
     wfirst, wpf, wnxt, wslot) = _run_router(xf, Wr, br)
    pos = pos2.reshape(T)

    sc_scatter, sc_gather = _sc_permute_kernels()
    xs = sc_scatter(xf, pos)
    ys = _run_ffn(wblk, wexp, wval, wstart, wend, wfirst, wpf, wnxt, wslot,
                  xs, W1, b1, W2, b2)
    outf = sc_gather(ys, pos)

    return outf.reshape(x.shape), loss2.reshape(())
